# baseline (speedup 1.0000x reference)
import functools

import jax
import jax.numpy as jnp
from jax import lax
from jax.experimental import pallas as pl
from jax.experimental.pallas import tpu as pltpu
from jax.experimental.pallas import tpu_sc as plsc

N = 10000
E = 320000
D = 128
NB = 79
NP = NB * 128
NC = 2
NS = 16
NW = NC * NS
ET = E // 128
ETB = ET // NW
EXW = ET - ETB * NW
EMAX = (ETB + 1) * 128

_mesh = plsc.VectorSubcoreMesh(core_axis_name="c", subcore_axis_name="s")


def _edge_slab(wid):
    ntiles = ETB + jnp.where(wid >= NW - EXW, 1, 0).astype(jnp.int32)
    off = 128 * (ETB * wid + jnp.maximum(wid - (NW - EXW), 0))
    return pl.multiple_of(off, 128), ntiles


@functools.partial(
    pl.kernel,
    mesh=_mesh,
    out_type=jax.ShapeDtypeStruct((NW, NP), jnp.float32),
    scratch_types=[
        pltpu.VMEM((2, EMAX), jnp.int32),
        pltpu.VMEM((NP,), jnp.float32),
        pltpu.SemaphoreType.DMA,
    ],
    compiler_params=pltpu.CompilerParams(needs_layout_passes=False),
)
def _sc_degree(ei_hbm, out_hbm, ei_v, acc_v, sem):
    wid = lax.axis_index("s") * NC + lax.axis_index("c")
    off, ntiles = _edge_slab(wid)
    cp = pltpu.async_copy(ei_hbm.at[:, pl.ds(off, EMAX)], ei_v, sem)

    zeros = jnp.zeros((16,), jnp.float32)

    def zbody(i, carry):
        base = i * 128
        for k in range(8):
            acc_v[pl.ds(base + k * 16, 16)] = zeros
        return carry

    lax.fori_loop(0, NP // 128, zbody, 0)
    cp.wait()

    ones = jnp.ones((16,), jnp.float32)

    def body(i, carry):
        base = i * 128
        for k in range(8):
            idx = ei_v[1, pl.ds(base + k * 16, 16)]
            plsc.addupdate_scatter(acc_v, [idx], ones)
        return carry

    lax.fori_loop(0, ntiles, body, 0)
    pltpu.sync_copy(acc_v, out_hbm.at[wid])


@functools.partial(
    pl.kernel,
    mesh=_mesh,
    out_type=jax.ShapeDtypeStruct((NW, NP), jnp.float32),
    scratch_types=[
        pltpu.VMEM((2, EMAX), jnp.int32),
        pltpu.VMEM((N,), jnp.float32),
        pltpu.VMEM((NP,), jnp.float32),
        pltpu.SemaphoreType.DMA,
    ],
    compiler_params=pltpu.CompilerParams(needs_layout_passes=False),
)
def _sc_aggregate(ei_hbm, v_hbm, out_hbm, ei_v, vv, acc_v, sem):
    wid = lax.axis_index("s") * NC + lax.axis_index("c")
    off, ntiles = _edge_slab(wid)
    c1 = pltpu.async_copy(ei_hbm.at[:, pl.ds(off, EMAX)], ei_v, sem)
    c2 = pltpu.async_copy(v_hbm, vv, sem)

    zeros = jnp.zeros((16,), jnp.float32)

    def zbody(i, carry):
        base = i * 128
        for k in range(8):
            acc_v[pl.ds(base + k * 16, 16)] = zeros
        return carry

    lax.fori_loop(0, NP // 128, zbody, 0)
    c1.wait()
    c2.wait()

    def gathers(t):
        base = t * 128
        return [
            plsc.load_gather(vv, [ei_v[0, pl.ds(base + k * 16, 16)]])
            for k in range(8)
        ]

    def body(i, vals):
        nxt = gathers(jnp.minimum(i + 1, ETB))
        base = i * 128
        for k in range(8):
            d16 = ei_v[1, pl.ds(base + k * 16, 16)]
            plsc.addupdate_scatter(acc_v, [d16], vals[k])
        return nxt

    lax.fori_loop(0, ntiles, body, gathers(0))
    pltpu.sync_copy(acc_v, out_hbm.at[wid])


def _tc_xw_body(x_ref, w_ref, o_ref):
    r = lax.dot_general(
        x_ref[...], w_ref[...],
        dimension_numbers=(((1,), (0,)), ((), ())),
        preferred_element_type=jnp.float32,
    )
    o_ref[...] = r.reshape(N)


_tc_xw = pl.pallas_call(
    _tc_xw_body,
    out_shape=jax.ShapeDtypeStruct((N,), jnp.float32),
)


def _tc_prep2_body(cnt_ref, xw_ref, v_ref, dinv_ref):
    deg = jnp.sum(cnt_ref[...], axis=0)[:N] + 1.0
    dinv = lax.rsqrt(deg)
    dinv_ref[...] = dinv
    v_ref[...] = xw_ref[...] * dinv


_tc_prep2 = pl.pallas_call(
    _tc_prep2_body,
    out_shape=[
        jax.ShapeDtypeStruct((N,), jnp.float32),
        jax.ShapeDtypeStruct((N,), jnp.float32),
    ],
)


def _tc_fin_body(p_ref, v_ref, dinv_ref, o_ref):
    o_ref[...] = dinv_ref[...] * (jnp.sum(p_ref[...], axis=0)[:N] + v_ref[...])


_tc_fin = pl.pallas_call(
    _tc_fin_body,
    out_shape=jax.ShapeDtypeStruct((N,), jnp.float32),
)


def kernel(x, edge_index, W):
    xw = _tc_xw(x, W)
    counts = _sc_degree(edge_index)
    v, dinv = _tc_prep2(counts, xw)
    parts = _sc_aggregate(edge_index, v)
    return _tc_fin(parts, v, dinv)


# speedup vs baseline: 1.0098x; 1.0098x over previous
import functools

import jax
import jax.numpy as jnp
from jax import lax
from jax.experimental import pallas as pl
from jax.experimental.pallas import tpu as pltpu
from jax.experimental.pallas import tpu_sc as plsc

N = 10000
E = 320000
D = 128
NB = 79
NP = NB * 128
NC = 2
NS = 16
NW = NC * NS
ET = E // 128
ETB = ET // NW
EXW = ET - ETB * NW
EMAX = (ETB + 1) * 128

_mesh = plsc.VectorSubcoreMesh(core_axis_name="c", subcore_axis_name="s")


def _edge_slab(wid):
    ntiles = ETB + jnp.where(wid >= NW - EXW, 1, 0).astype(jnp.int32)
    off = 128 * (ETB * wid + jnp.maximum(wid - (NW - EXW), 0))
    return pl.multiple_of(off, 128), ntiles


@functools.partial(
    pl.kernel,
    mesh=_mesh,
    out_type=jax.ShapeDtypeStruct((NW, NP), jnp.float32),
    scratch_types=[
        pltpu.VMEM((2, EMAX), jnp.int32),
        pltpu.VMEM((NP,), jnp.float32),
        pltpu.SemaphoreType.DMA,
    ],
    compiler_params=pltpu.CompilerParams(needs_layout_passes=False),
)
def _sc_degree(ei_hbm, out_hbm, ei_v, acc_v, sem):
    wid = lax.axis_index("s") * NC + lax.axis_index("c")
    off, ntiles = _edge_slab(wid)
    cp = pltpu.async_copy(ei_hbm.at[:, pl.ds(off, EMAX)], ei_v, sem)

    zeros = jnp.zeros((16,), jnp.float32)

    def zbody(i, carry):
        base = i * 128
        for k in range(8):
            acc_v[pl.ds(base + k * 16, 16)] = zeros
        return carry

    lax.fori_loop(0, NP // 128, zbody, 0)
    cp.wait()

    ones = jnp.ones((16,), jnp.float32)

    def body(i, carry):
        base = i * 128
        for k in range(8):
            idx = ei_v[1, pl.ds(base + k * 16, 16)]
            plsc.addupdate_scatter(acc_v, [idx], ones)
        return carry

    lax.fori_loop(0, ntiles, body, 0)
    pltpu.sync_copy(acc_v, out_hbm.at[wid])


@functools.partial(
    pl.kernel,
    mesh=_mesh,
    out_type=jax.ShapeDtypeStruct((NW, NP), jnp.float32),
    scratch_types=[
        pltpu.VMEM((2, EMAX), jnp.int32),
        pltpu.VMEM((N,), jnp.float32),
        pltpu.VMEM((NP,), jnp.float32),
        pltpu.SemaphoreType.DMA,
    ],
    compiler_params=pltpu.CompilerParams(needs_layout_passes=False),
)
def _sc_aggregate(ei_hbm, v_hbm, out_hbm, ei_v, vv, acc_v, sem):
    wid = lax.axis_index("s") * NC + lax.axis_index("c")
    off, ntiles = _edge_slab(wid)
    c1 = pltpu.async_copy(ei_hbm.at[:, pl.ds(off, EMAX)], ei_v, sem)
    c2 = pltpu.async_copy(v_hbm, vv, sem)

    zeros = jnp.zeros((16,), jnp.float32)

    def zbody(i, carry):
        base = i * 128
        for k in range(8):
            acc_v[pl.ds(base + k * 16, 16)] = zeros
        return carry

    lax.fori_loop(0, NP // 128, zbody, 0)
    c1.wait()
    c2.wait()

    def body(i, carry):
        base = i * 128
        vals = []
        for k in range(8):
            s16 = ei_v[0, pl.ds(base + k * 16, 16)]
            vals.append(plsc.load_gather(vv, [s16]))
        for k in range(8):
            d16 = ei_v[1, pl.ds(base + k * 16, 16)]
            plsc.addupdate_scatter(acc_v, [d16], vals[k])
        return carry

    lax.fori_loop(0, ntiles, body, 0)
    pltpu.sync_copy(acc_v, out_hbm.at[wid])


def _tc_xw_body(x_ref, w_ref, o_ref):
    r = lax.dot_general(
        x_ref[...], w_ref[...],
        dimension_numbers=(((1,), (0,)), ((), ())),
        preferred_element_type=jnp.float32,
    )
    o_ref[...] = r.reshape(N)


_tc_xw = pl.pallas_call(
    _tc_xw_body,
    out_shape=jax.ShapeDtypeStruct((N,), jnp.float32),
)


def _tc_prep2_body(cnt_ref, xw_ref, v_ref, dinv_ref):
    deg = jnp.sum(cnt_ref[...], axis=0)[:N] + 1.0
    dinv = lax.rsqrt(deg)
    dinv_ref[...] = dinv
    v_ref[...] = xw_ref[...] * dinv


_tc_prep2 = pl.pallas_call(
    _tc_prep2_body,
    out_shape=[
        jax.ShapeDtypeStruct((N,), jnp.float32),
        jax.ShapeDtypeStruct((N,), jnp.float32),
    ],
)


def _tc_fin_body(p_ref, v_ref, dinv_ref, o_ref):
    o_ref[...] = dinv_ref[...] * (jnp.sum(p_ref[...], axis=0)[:N] + v_ref[...])


_tc_fin = pl.pallas_call(
    _tc_fin_body,
    out_shape=jax.ShapeDtypeStruct((N,), jnp.float32),
)


def kernel(x, edge_index, W):
    xw = _tc_xw(x, W)
    counts = _sc_degree(edge_index)
    v, dinv = _tc_prep2(counts, xw)
    parts = _sc_aggregate(edge_index, v)
    return _tc_fin(parts, v, dinv)
